# baseline (speedup 1.0000x reference)
import jax
import jax.numpy as jnp
from jax import lax
from jax.experimental import pallas as pl
from jax.experimental.pallas import tpu as pltpu
from jax.experimental.pallas import tpu_sc as plsc

N = 10000
E = 320000
D = 128
DH = D // 2
NC, NS = 2, 16
NW = NC * NS
NP = 10240
RPT = NP // NS
NACC = 10112
RPA = NACC // NS
CH = 120
DEPTH = 4
NCH_AGG = 168
E_PAD = NS * NCH_AGG * CH
NCH_DEG = E_PAD // NW // CH

f32 = jnp.float32
_mesh = plsc.VectorSubcoreMesh(core_axis_name="c", subcore_axis_name="s")
_sc_params = pltpu.CompilerParams(use_tc_tiling_on_sc=False)


def _deg_body(dst_hbm, deg_hbm, idx_v, ones_v, zb_v, acc_sh):
    c = lax.axis_index("c")
    s = lax.axis_index("s")

    def zz(i, _):
        zb_v[pl.ds(i * 16, 16)] = jnp.zeros((16,), f32)
        return 0
    lax.fori_loop(0, RPT // 16, zz, 0)

    def oo(i, _):
        ones_v[pl.ds(i * 16, 16)] = jnp.ones((16,), f32)
        return 0
    lax.fori_loop(0, CH // 16, oo, 0)

    pltpu.sync_copy(zb_v, acc_sh.at[pl.ds(s * RPT, RPT)])
    w = s * NC + c
    pltpu.sync_copy(dst_hbm.at[w], idx_v)
    plsc.subcore_barrier()

    def body(j, _):
        pltpu.sync_copy(ones_v, acc_sh.at[idx_v.at[j]], add=True)
        return 0
    lax.fori_loop(0, NCH_DEG, body, 0)
    plsc.subcore_barrier()

    pltpu.sync_copy(acc_sh.at[pl.ds(s * RPT, RPT)], zb_v)
    pltpu.sync_copy(zb_v, deg_hbm.at[c, pl.ds(s * RPT, RPT)])


_deg_call = pl.kernel(
    _deg_body,
    out_type=jax.ShapeDtypeStruct((NC, NP), f32),
    mesh=_mesh,
    scratch_types=[
        pltpu.VMEM((NCH_DEG, CH), jnp.int32),
        pltpu.VMEM((CH,), f32),
        pltpu.VMEM((RPT,), f32),
        pltpu.VMEM_SHARED((NP,), f32),
    ],
    compiler_params=_sc_params,
)


def _agg_body(y_hbm, src_hbm, dst_hbm, out_hbm, isrc_v, idst_v,
              r0, r1, r2, r3, zb_v, acc_sh,
              g0, g1, g2, g3):
    c = lax.axis_index("c")
    s = lax.axis_index("s")
    rows = (r0, r1, r2, r3)
    gsem = (g0, g1, g2, g3)

    def gather(j, b):
        pltpu.async_copy(y_hbm.at[c].at[isrc_v.at[j]], rows[b], gsem[b])

    def gwait(j, b):
        pltpu.make_async_copy(
            y_hbm.at[c].at[isrc_v.at[j]], rows[b], gsem[b]).wait()

    def scatter(j, b):
        pltpu.sync_copy(rows[b], acc_sh.at[idst_v.at[j]], add=True)

    LEAD = DEPTH // 2
    pltpu.sync_copy(src_hbm.at[s], isrc_v)
    pltpu.sync_copy(dst_hbm.at[s], idst_v)
    for b in range(LEAD):
        gather(b, b)

    def zz(t, _):
        zb_v[t // 4, pl.ds((t % 4) * 16, 16)] = jnp.zeros((16,), f32)
        return 0
    lax.fori_loop(0, CH * 4, zz, 0)

    def zc(k, _):
        pltpu.sync_copy(zb_v, acc_sh.at[pl.ds(s * RPA + k * CH, CH)])
        return 0
    lax.fori_loop(0, RPA // CH, zc, 0)
    if RPA % CH:
        pltpu.sync_copy(zb_v.at[pl.ds(0, RPA % CH)],
                        acc_sh.at[pl.ds(s * RPA + (RPA // CH) * CH, RPA % CH)])
    plsc.subcore_barrier()

    for j in range(LEAD):
        gwait(j, j)
        scatter(j, j)
        gather(j + LEAD, j + LEAD)

    def body(jj, _):
        jb = LEAD + jj * DEPTH
        for t in range(DEPTH):
            j = jb + t
            b = (LEAD + t) % DEPTH
            gwait(j, b)
            scatter(j, b)
            gather(j + LEAD, t % DEPTH)
        return 0
    lax.fori_loop(0, (NCH_AGG - 2 * LEAD) // DEPTH, body, 0)

    for t in range(LEAD):
        j = NCH_AGG - LEAD + t
        b = j % DEPTH
        gwait(j, b)
        scatter(j, b)
    plsc.subcore_barrier()

    pltpu.sync_copy(acc_sh.at[pl.ds(s * RPA, RPA)],
                    out_hbm.at[c, pl.ds(s * RPA, RPA)])


_agg_call = pl.kernel(
    _agg_body,
    out_type=jax.ShapeDtypeStruct((NC, NACC, DH), f32),
    mesh=_mesh,
    scratch_types=[
        pltpu.VMEM((NCH_AGG, CH), jnp.int32),
        pltpu.VMEM((NCH_AGG, CH), jnp.int32),
    ] + [pltpu.VMEM((CH, DH), f32) for _ in range(DEPTH + 1)] + [
        pltpu.VMEM_SHARED((NACC, DH), f32),
    ] + [pltpu.SemaphoreType.DMA for _ in range(DEPTH)],
    compiler_params=_sc_params,
)


BN = 1024
GRID = NP // BN


def _tc1_body(deg_ref, x_ref, w_ref, dinv_ref, y_ref):
    deg = deg_ref[0, :] + deg_ref[1, :] + 1.0
    dinv = lax.rsqrt(deg)
    dinv_ref[...] = dinv[:, None]
    y = jnp.dot(x_ref[...], w_ref[...],
                preferred_element_type=f32) * dinv[:, None]
    y_ref[0] = y[:, :DH]
    y_ref[1] = y[:, DH:]


def _tc1(deg2, x, W1):
    return pl.pallas_call(
        _tc1_body,
        grid=(GRID,),
        in_specs=[
            pl.BlockSpec((NC, BN), lambda i: (0, i)),
            pl.BlockSpec((BN, D), lambda i: (i, 0)),
            pl.BlockSpec((D, D), lambda i: (0, 0)),
        ],
        out_specs=[
            pl.BlockSpec((BN, 1), lambda i: (i, 0)),
            pl.BlockSpec((NC, BN, DH), lambda i: (0, i, 0)),
        ],
        out_shape=[
            jax.ShapeDtypeStruct((N, 1), f32),
            jax.ShapeDtypeStruct((NC, N, DH), f32),
        ],
    )(deg2, x, W1)


def _tcmid_body(p_ref, y_ref, dinv_ref, b_ref, w_ref, o_ref):
    h = jnp.concatenate(
        [p_ref[0] + y_ref[0], p_ref[1] + y_ref[1]], axis=1)
    h = h * dinv_ref[...] + b_ref[...]
    h = jnp.maximum(h, 0.0)
    o = jnp.dot(h, w_ref[...], preferred_element_type=f32) * dinv_ref[...]
    o_ref[0] = o[:, :DH]
    o_ref[1] = o[:, DH:]


def _tcmid(p, y, dinv, b, Wn):
    return pl.pallas_call(
        _tcmid_body,
        grid=(GRID,),
        in_specs=[
            pl.BlockSpec((NC, BN, DH), lambda i: (0, i, 0)),
            pl.BlockSpec((NC, BN, DH), lambda i: (0, i, 0)),
            pl.BlockSpec((BN, 1), lambda i: (i, 0)),
            pl.BlockSpec((1, D), lambda i: (0, 0)),
            pl.BlockSpec((D, D), lambda i: (0, 0)),
        ],
        out_specs=pl.BlockSpec((NC, BN, DH), lambda i: (0, i, 0)),
        out_shape=jax.ShapeDtypeStruct((NC, N, DH), f32),
    )(p, y, dinv, b, Wn)


def _tcfin_body(p_ref, y_ref, dinv_ref, b_ref, wl_ref, bl_ref, o_ref):
    h = jnp.concatenate(
        [p_ref[0] + y_ref[0], p_ref[1] + y_ref[1]], axis=1)
    h = h * dinv_ref[...] + b_ref[...]
    o_ref[...] = jnp.dot(h, wl_ref[...],
                         preferred_element_type=f32) + bl_ref[...]


def _tcfin(p, y, dinv, b, Wl, bl):
    return pl.pallas_call(
        _tcfin_body,
        grid=(GRID,),
        in_specs=[
            pl.BlockSpec((NC, BN, DH), lambda i: (0, i, 0)),
            pl.BlockSpec((NC, BN, DH), lambda i: (0, i, 0)),
            pl.BlockSpec((BN, 1), lambda i: (i, 0)),
            pl.BlockSpec((1, D), lambda i: (0, 0)),
            pl.BlockSpec((D, 1), lambda i: (0, 0)),
            pl.BlockSpec((1, 1), lambda i: (0, 0)),
        ],
        out_specs=pl.BlockSpec((BN, 1), lambda i: (i, 0)),
        out_shape=jax.ShapeDtypeStruct((N, 1), f32),
    )(p, y, dinv, b, Wl, bl)


@jax.jit
def _run(x, srcw, dsts, dstw, W1, b1, W2, b2, W3, b3, Wl, bl):
    deg2 = _deg_call(dstw)
    dinv, y1 = _tc1(deg2, x, W1)
    p = _agg_call(y1, srcw, dsts)
    y2 = _tcmid(p, y1, dinv, b1, W2)
    p = _agg_call(y2, srcw, dsts)
    y3 = _tcmid(p, y2, dinv, b2, W3)
    p = _agg_call(y3, srcw, dsts)
    return _tcfin(p, y3, dinv, b3, Wl, bl)


def kernel(x, edge_index, W1, b1, W2, b2, W3, b3, Wl, bl):
    ei = edge_index.astype(jnp.int32)
    pad = E_PAD - E
    src_p = jnp.concatenate([ei[0], jnp.zeros((pad,), jnp.int32)])
    dst_p = jnp.concatenate([ei[1], jnp.full((pad,), N, jnp.int32)])
    srcw = src_p.reshape(NS, NCH_AGG, CH)
    dsts = dst_p.reshape(NS, NCH_AGG, CH)
    dstw = dst_p.reshape(NW, NCH_DEG, CH)
    return _run(x, srcw, dsts, dstw, W1, b1.reshape(1, D), W2,
                b2.reshape(1, D), W3, b3.reshape(1, D), Wl, bl.reshape(1, 1))


# speedup vs baseline: 1.1364x; 1.1364x over previous
import jax
import jax.numpy as jnp
from jax import lax
from jax.experimental import pallas as pl
from jax.experimental.pallas import tpu as pltpu
from jax.experimental.pallas import tpu_sc as plsc

N = 10000
E = 320000
D = 128
DH = D // 2
NC, NS = 2, 16
NW = NC * NS
NP = 10240
RPT = NP // NS
NACC = 10112
RPA = NACC // NS
CH = 112
DEPTH = 6
NCH_AGG = 180
E_PAD = NS * NCH_AGG * CH
NCH_DEG = E_PAD // NW // CH

f32 = jnp.float32
_mesh = plsc.VectorSubcoreMesh(core_axis_name="c", subcore_axis_name="s")
_sc_params = pltpu.CompilerParams(use_tc_tiling_on_sc=False)


def _deg_body(dst_hbm, deg_hbm, idx_v, ones_v, zb_v, acc_sh):
    c = lax.axis_index("c")
    s = lax.axis_index("s")

    def zz(i, _):
        zb_v[pl.ds(i * 16, 16)] = jnp.zeros((16,), f32)
        return 0
    lax.fori_loop(0, RPT // 16, zz, 0)

    def oo(i, _):
        ones_v[pl.ds(i * 16, 16)] = jnp.ones((16,), f32)
        return 0
    lax.fori_loop(0, CH // 16, oo, 0)

    pltpu.sync_copy(zb_v, acc_sh.at[pl.ds(s * RPT, RPT)])
    w = s * NC + c
    pltpu.sync_copy(dst_hbm.at[w], idx_v)
    plsc.subcore_barrier()

    def body(j, _):
        pltpu.sync_copy(ones_v, acc_sh.at[idx_v.at[j]], add=True)
        return 0
    lax.fori_loop(0, NCH_DEG, body, 0)
    plsc.subcore_barrier()

    pltpu.sync_copy(acc_sh.at[pl.ds(s * RPT, RPT)], zb_v)
    pltpu.sync_copy(zb_v, deg_hbm.at[c, pl.ds(s * RPT, RPT)])


_deg_call = pl.kernel(
    _deg_body,
    out_type=jax.ShapeDtypeStruct((NC, NP), f32),
    mesh=_mesh,
    scratch_types=[
        pltpu.VMEM((NCH_DEG, CH), jnp.int32),
        pltpu.VMEM((CH,), f32),
        pltpu.VMEM((RPT,), f32),
        pltpu.VMEM_SHARED((NP,), f32),
    ],
    compiler_params=_sc_params,
)


def _agg_body(y_hbm, src_hbm, dst_hbm, out_hbm, isrc_v, idst_v,
              r0, r1, r2, r3, r4, r5, zb_v, acc_sh,
              g0, g1, g2, g3, g4, g5):
    c = lax.axis_index("c")
    s = lax.axis_index("s")
    rows = (r0, r1, r2, r3, r4, r5)
    gsem = (g0, g1, g2, g3, g4, g5)

    def gather(j, b):
        pltpu.async_copy(y_hbm.at[c].at[isrc_v.at[j]], rows[b], gsem[b])

    def gwait(j, b):
        pltpu.make_async_copy(
            y_hbm.at[c].at[isrc_v.at[j]], rows[b], gsem[b]).wait()

    def scatter(j, b):
        pltpu.sync_copy(rows[b], acc_sh.at[idst_v.at[j]], add=True)

    LEAD = DEPTH // 2
    pltpu.sync_copy(src_hbm.at[s], isrc_v)
    pltpu.sync_copy(dst_hbm.at[s], idst_v)
    for b in range(LEAD):
        gather(b, b)

    def zz(t, _):
        zb_v[t // 4, pl.ds((t % 4) * 16, 16)] = jnp.zeros((16,), f32)
        return 0
    lax.fori_loop(0, CH * 4, zz, 0)

    def zc(k, _):
        pltpu.sync_copy(zb_v, acc_sh.at[pl.ds(s * RPA + k * CH, CH)])
        return 0
    lax.fori_loop(0, RPA // CH, zc, 0)
    if RPA % CH:
        pltpu.sync_copy(zb_v.at[pl.ds(0, RPA % CH)],
                        acc_sh.at[pl.ds(s * RPA + (RPA // CH) * CH, RPA % CH)])
    plsc.subcore_barrier()

    for j in range(LEAD):
        gwait(j, j)
        scatter(j, j)
        gather(j + LEAD, j + LEAD)

    def body(jj, _):
        jb = LEAD + jj * DEPTH
        for t in range(DEPTH):
            j = jb + t
            b = (LEAD + t) % DEPTH
            gwait(j, b)
            scatter(j, b)
            gather(j + LEAD, t % DEPTH)
        return 0
    lax.fori_loop(0, (NCH_AGG - 2 * LEAD) // DEPTH, body, 0)

    for t in range(LEAD):
        j = NCH_AGG - LEAD + t
        b = j % DEPTH
        gwait(j, b)
        scatter(j, b)
    plsc.subcore_barrier()

    pltpu.sync_copy(acc_sh.at[pl.ds(s * RPA, RPA)],
                    out_hbm.at[c, pl.ds(s * RPA, RPA)])


_agg_call = pl.kernel(
    _agg_body,
    out_type=jax.ShapeDtypeStruct((NC, NACC, DH), f32),
    mesh=_mesh,
    scratch_types=[
        pltpu.VMEM((NCH_AGG, CH), jnp.int32),
        pltpu.VMEM((NCH_AGG, CH), jnp.int32),
    ] + [pltpu.VMEM((CH, DH), f32) for _ in range(DEPTH + 1)] + [
        pltpu.VMEM_SHARED((NACC, DH), f32),
    ] + [pltpu.SemaphoreType.DMA for _ in range(DEPTH)],
    compiler_params=_sc_params,
)


BN = 1024
GRID = NP // BN


def _mm_body(x_ref, w_ref, o_ref):
    o_ref[...] = jnp.dot(x_ref[...], w_ref[...], preferred_element_type=f32)


def _mm(x, W1):
    return pl.pallas_call(
        _mm_body,
        grid=(GRID,),
        in_specs=[
            pl.BlockSpec((BN, D), lambda i: (i, 0)),
            pl.BlockSpec((D, D), lambda i: (0, 0)),
        ],
        out_specs=pl.BlockSpec((BN, D), lambda i: (i, 0)),
        out_shape=jax.ShapeDtypeStruct((N, D), f32),
    )(x, W1)


def _tc1_body(deg_ref, xw_ref, dinv_ref, y_ref):
    deg = deg_ref[0, :] + deg_ref[1, :] + 1.0
    dinv = lax.rsqrt(deg)
    dinv_ref[...] = dinv[:, None]
    y = xw_ref[...] * dinv[:, None]
    y_ref[0] = y[:, :DH]
    y_ref[1] = y[:, DH:]


def _tc1(deg2, xw):
    return pl.pallas_call(
        _tc1_body,
        grid=(GRID,),
        in_specs=[
            pl.BlockSpec((NC, BN), lambda i: (0, i)),
            pl.BlockSpec((BN, D), lambda i: (i, 0)),
        ],
        out_specs=[
            pl.BlockSpec((BN, 1), lambda i: (i, 0)),
            pl.BlockSpec((NC, BN, DH), lambda i: (0, i, 0)),
        ],
        out_shape=[
            jax.ShapeDtypeStruct((N, 1), f32),
            jax.ShapeDtypeStruct((NC, N, DH), f32),
        ],
    )(deg2, xw)


def _tcmid_body(p_ref, y_ref, dinv_ref, b_ref, w_ref, o_ref):
    h = jnp.concatenate(
        [p_ref[0] + y_ref[0], p_ref[1] + y_ref[1]], axis=1)
    h = h * dinv_ref[...] + b_ref[...]
    h = jnp.maximum(h, 0.0)
    o = jnp.dot(h, w_ref[...], preferred_element_type=f32) * dinv_ref[...]
    o_ref[0] = o[:, :DH]
    o_ref[1] = o[:, DH:]


def _tcmid(p, y, dinv, b, Wn):
    return pl.pallas_call(
        _tcmid_body,
        grid=(GRID,),
        in_specs=[
            pl.BlockSpec((NC, BN, DH), lambda i: (0, i, 0)),
            pl.BlockSpec((NC, BN, DH), lambda i: (0, i, 0)),
            pl.BlockSpec((BN, 1), lambda i: (i, 0)),
            pl.BlockSpec((1, D), lambda i: (0, 0)),
            pl.BlockSpec((D, D), lambda i: (0, 0)),
        ],
        out_specs=pl.BlockSpec((NC, BN, DH), lambda i: (0, i, 0)),
        out_shape=jax.ShapeDtypeStruct((NC, N, DH), f32),
    )(p, y, dinv, b, Wn)


def _tcfin_body(p_ref, y_ref, dinv_ref, b_ref, wl_ref, bl_ref, o_ref):
    h = jnp.concatenate(
        [p_ref[0] + y_ref[0], p_ref[1] + y_ref[1]], axis=1)
    h = h * dinv_ref[...] + b_ref[...]
    o_ref[...] = jnp.dot(h, wl_ref[...],
                         preferred_element_type=f32) + bl_ref[...]


def _tcfin(p, y, dinv, b, Wl, bl):
    return pl.pallas_call(
        _tcfin_body,
        grid=(GRID,),
        in_specs=[
            pl.BlockSpec((NC, BN, DH), lambda i: (0, i, 0)),
            pl.BlockSpec((NC, BN, DH), lambda i: (0, i, 0)),
            pl.BlockSpec((BN, 1), lambda i: (i, 0)),
            pl.BlockSpec((1, D), lambda i: (0, 0)),
            pl.BlockSpec((D, 1), lambda i: (0, 0)),
            pl.BlockSpec((1, 1), lambda i: (0, 0)),
        ],
        out_specs=pl.BlockSpec((BN, 1), lambda i: (i, 0)),
        out_shape=jax.ShapeDtypeStruct((N, 1), f32),
    )(p, y, dinv, b, Wl, bl)


@jax.jit
def _run(x, srcw, dsts, dstw, W1, b1, W2, b2, W3, b3, Wl, bl):
    deg2 = _deg_call(dstw)
    xw1 = _mm(x, W1)
    dinv, y1 = _tc1(deg2, xw1)
    p = _agg_call(y1, srcw, dsts)
    y2 = _tcmid(p, y1, dinv, b1, W2)
    p = _agg_call(y2, srcw, dsts)
    y3 = _tcmid(p, y2, dinv, b2, W3)
    p = _agg_call(y3, srcw, dsts)
    return _tcfin(p, y3, dinv, b3, Wl, bl)


def kernel(x, edge_index, W1, b1, W2, b2, W3, b3, Wl, bl):
    ei = edge_index.astype(jnp.int32)
    pad = E_PAD - E
    src_p = jnp.concatenate([ei[0], jnp.zeros((pad,), jnp.int32)])
    dst_p = jnp.concatenate([ei[1], jnp.full((pad,), N, jnp.int32)])
    srcw = src_p.reshape(NS, NCH_AGG, CH)
    dsts = dst_p.reshape(NS, NCH_AGG, CH)
    dstw = dst_p.reshape(NW, NCH_DEG, CH)
    return _run(x, srcw, dsts, dstw, W1, b1.reshape(1, D), W2,
                b2.reshape(1, D), W3, b3.reshape(1, D), Wl, bl.reshape(1, 1))


# speedup vs baseline: 1.1517x; 1.0134x over previous
import jax
import jax.numpy as jnp
from jax import lax
from jax.experimental import pallas as pl
from jax.experimental.pallas import tpu as pltpu
from jax.experimental.pallas import tpu_sc as plsc

N = 10000
E = 320000
D = 128
DH = D // 2
NC, NS = 2, 16
NW = NC * NS
NP = 10240
RPT = NP // NS
NACC = 10112
RPA = NACC // NS
CH = 112
DEPTH = 6
NCH_AGG = 180
E_PAD = NS * NCH_AGG * CH
NCH_DEG = E_PAD // NW // CH

f32 = jnp.float32
_mesh = plsc.VectorSubcoreMesh(core_axis_name="c", subcore_axis_name="s")
_sc_params = pltpu.CompilerParams(use_tc_tiling_on_sc=False)


def _deg_body(dst_hbm, deg_hbm, idx_v, ones_v, zb_v, acc_sh):
    c = lax.axis_index("c")
    s = lax.axis_index("s")

    def zz(i, _):
        zb_v[pl.ds(i * 16, 16)] = jnp.zeros((16,), f32)
        return 0
    lax.fori_loop(0, RPT // 16, zz, 0)

    def oo(i, _):
        ones_v[pl.ds(i * 16, 16)] = jnp.ones((16,), f32)
        return 0
    lax.fori_loop(0, CH // 16, oo, 0)

    pltpu.sync_copy(zb_v, acc_sh.at[pl.ds(s * RPT, RPT)])
    w = s * NC + c
    pltpu.sync_copy(dst_hbm.at[w], idx_v)
    plsc.subcore_barrier()

    def body(j, _):
        pltpu.sync_copy(ones_v, acc_sh.at[idx_v.at[j]], add=True)
        return 0
    lax.fori_loop(0, NCH_DEG, body, 0)
    plsc.subcore_barrier()

    pltpu.sync_copy(acc_sh.at[pl.ds(s * RPT, RPT)], zb_v)
    pltpu.sync_copy(zb_v, deg_hbm.at[c, pl.ds(s * RPT, RPT)])


_deg_call = pl.kernel(
    _deg_body,
    out_type=jax.ShapeDtypeStruct((NC, NP), f32),
    mesh=_mesh,
    scratch_types=[
        pltpu.VMEM((NCH_DEG, CH), jnp.int32),
        pltpu.VMEM((CH,), f32),
        pltpu.VMEM((RPT,), f32),
        pltpu.VMEM_SHARED((NP,), f32),
    ],
    compiler_params=_sc_params,
)


def _agg_body(y_hbm, src_hbm, dst_hbm, out_hbm, isrc_v, idst_v,
              r0, r1, r2, r3, r4, r5, zb_v, acc_sh,
              g0, g1, g2, g3, g4, g5):
    c = lax.axis_index("c")
    s = lax.axis_index("s")
    rows = (r0, r1, r2, r3, r4, r5)
    gsem = (g0, g1, g2, g3, g4, g5)

    def gather(j, b):
        pltpu.async_copy(y_hbm.at[c].at[isrc_v.at[j]], rows[b], gsem[b])

    def gwait(j, b):
        pltpu.make_async_copy(
            y_hbm.at[c].at[isrc_v.at[j]], rows[b], gsem[b]).wait()

    def scatter(j, b):
        pltpu.sync_copy(rows[b], acc_sh.at[idst_v.at[j]], add=True)

    LEAD = DEPTH // 2
    pltpu.sync_copy(src_hbm.at[s], isrc_v)
    pltpu.sync_copy(dst_hbm.at[s], idst_v)
    for b in range(LEAD):
        gather(b, b)

    def zz(t, _):
        zb_v[t // 4, pl.ds((t % 4) * 16, 16)] = jnp.zeros((16,), f32)
        return 0
    lax.fori_loop(0, CH * 4, zz, 0)

    def zc(k, _):
        pltpu.sync_copy(zb_v, acc_sh.at[pl.ds(s * RPA + k * CH, CH)])
        return 0
    lax.fori_loop(0, RPA // CH, zc, 0)
    if RPA % CH:
        pltpu.sync_copy(zb_v.at[pl.ds(0, RPA % CH)],
                        acc_sh.at[pl.ds(s * RPA + (RPA // CH) * CH, RPA % CH)])
    plsc.subcore_barrier()

    for j in range(LEAD):
        gwait(j, j)
        scatter(j, j)
        gather(j + LEAD, j + LEAD)

    def body(jj, _):
        jb = LEAD + jj * DEPTH
        for t in range(DEPTH):
            j = jb + t
            b = (LEAD + t) % DEPTH
            gwait(j, b)
            scatter(j, b)
            gather(j + LEAD, t % DEPTH)
        return 0
    lax.fori_loop(0, (NCH_AGG - 2 * LEAD) // DEPTH, body, 0)

    for t in range(LEAD):
        j = NCH_AGG - LEAD + t
        b = j % DEPTH
        gwait(j, b)
        scatter(j, b)
    plsc.subcore_barrier()

    pltpu.sync_copy(acc_sh.at[pl.ds(s * RPA, RPA)],
                    out_hbm.at[c, pl.ds(s * RPA, RPA)])


_agg_call = pl.kernel(
    _agg_body,
    out_type=jax.ShapeDtypeStruct((NC, NACC, DH), f32),
    mesh=_mesh,
    scratch_types=[
        pltpu.VMEM((NCH_AGG, CH), jnp.int32),
        pltpu.VMEM((NCH_AGG, CH), jnp.int32),
    ] + [pltpu.VMEM((CH, DH), f32) for _ in range(DEPTH + 1)] + [
        pltpu.VMEM_SHARED((NACC, DH), f32),
    ] + [pltpu.SemaphoreType.DMA for _ in range(DEPTH)],
    compiler_params=_sc_params,
)


BN = 2048
GRID = NP // BN


def _mm_body(x_ref, w_ref, o_ref):
    o_ref[...] = jnp.dot(x_ref[...], w_ref[...], preferred_element_type=f32)


def _mm(x, W1):
    return pl.pallas_call(
        _mm_body,
        grid=(GRID,),
        in_specs=[
            pl.BlockSpec((BN, D), lambda i: (i, 0)),
            pl.BlockSpec((D, D), lambda i: (0, 0)),
        ],
        out_specs=pl.BlockSpec((BN, D), lambda i: (i, 0)),
        out_shape=jax.ShapeDtypeStruct((N, D), f32),
    )(x, W1)


def _tc1_body(deg_ref, xw_ref, dinv_ref, y_ref):
    deg = deg_ref[0, :] + deg_ref[1, :] + 1.0
    dinv = lax.rsqrt(deg)
    dinv_ref[...] = dinv[:, None]
    y = xw_ref[...] * dinv[:, None]
    y_ref[0] = y[:, :DH]
    y_ref[1] = y[:, DH:]


def _tc1(deg2, xw):
    return pl.pallas_call(
        _tc1_body,
        grid=(GRID,),
        in_specs=[
            pl.BlockSpec((NC, BN), lambda i: (0, i)),
            pl.BlockSpec((BN, D), lambda i: (i, 0)),
        ],
        out_specs=[
            pl.BlockSpec((BN, 1), lambda i: (i, 0)),
            pl.BlockSpec((NC, BN, DH), lambda i: (0, i, 0)),
        ],
        out_shape=[
            jax.ShapeDtypeStruct((N, 1), f32),
            jax.ShapeDtypeStruct((NC, N, DH), f32),
        ],
    )(deg2, xw)


def _tcmid_body(p_ref, y_ref, dinv_ref, b_ref, w_ref, o_ref):
    h = jnp.concatenate(
        [p_ref[0] + y_ref[0], p_ref[1] + y_ref[1]], axis=1)
    h = h * dinv_ref[...] + b_ref[...]
    h = jnp.maximum(h, 0.0)
    o = jnp.dot(h, w_ref[...], preferred_element_type=f32) * dinv_ref[...]
    o_ref[0] = o[:, :DH]
    o_ref[1] = o[:, DH:]


def _tcmid(p, y, dinv, b, Wn):
    return pl.pallas_call(
        _tcmid_body,
        grid=(GRID,),
        in_specs=[
            pl.BlockSpec((NC, BN, DH), lambda i: (0, i, 0)),
            pl.BlockSpec((NC, BN, DH), lambda i: (0, i, 0)),
            pl.BlockSpec((BN, 1), lambda i: (i, 0)),
            pl.BlockSpec((1, D), lambda i: (0, 0)),
            pl.BlockSpec((D, D), lambda i: (0, 0)),
        ],
        out_specs=pl.BlockSpec((NC, BN, DH), lambda i: (0, i, 0)),
        out_shape=jax.ShapeDtypeStruct((NC, N, DH), f32),
    )(p, y, dinv, b, Wn)


def _tcfin_body(p_ref, y_ref, dinv_ref, b_ref, wl_ref, bl_ref, o_ref):
    h = jnp.concatenate(
        [p_ref[0] + y_ref[0], p_ref[1] + y_ref[1]], axis=1)
    h = h * dinv_ref[...] + b_ref[...]
    o_ref[...] = jnp.dot(h, wl_ref[...],
                         preferred_element_type=f32) + bl_ref[...]


def _tcfin(p, y, dinv, b, Wl, bl):
    return pl.pallas_call(
        _tcfin_body,
        grid=(GRID,),
        in_specs=[
            pl.BlockSpec((NC, BN, DH), lambda i: (0, i, 0)),
            pl.BlockSpec((NC, BN, DH), lambda i: (0, i, 0)),
            pl.BlockSpec((BN, 1), lambda i: (i, 0)),
            pl.BlockSpec((1, D), lambda i: (0, 0)),
            pl.BlockSpec((D, 1), lambda i: (0, 0)),
            pl.BlockSpec((1, 1), lambda i: (0, 0)),
        ],
        out_specs=pl.BlockSpec((BN, 1), lambda i: (i, 0)),
        out_shape=jax.ShapeDtypeStruct((N, 1), f32),
    )(p, y, dinv, b, Wl, bl)


@jax.jit
def _run(x, srcw, dsts, dstw, W1, b1, W2, b2, W3, b3, Wl, bl):
    deg2 = _deg_call(dstw)
    xw1 = _mm(x, W1)
    dinv, y1 = _tc1(deg2, xw1)
    p = _agg_call(y1, srcw, dsts)
    y2 = _tcmid(p, y1, dinv, b1, W2)
    p = _agg_call(y2, srcw, dsts)
    y3 = _tcmid(p, y2, dinv, b2, W3)
    p = _agg_call(y3, srcw, dsts)
    return _tcfin(p, y3, dinv, b3, Wl, bl)


def kernel(x, edge_index, W1, b1, W2, b2, W3, b3, Wl, bl):
    ei = edge_index.astype(jnp.int32)
    pad = E_PAD - E
    src_p = jnp.concatenate([ei[0], jnp.zeros((pad,), jnp.int32)])
    dst_p = jnp.concatenate([ei[1], jnp.full((pad,), N, jnp.int32)])
    srcw = src_p.reshape(NS, NCH_AGG, CH)
    dsts = dst_p.reshape(NS, NCH_AGG, CH)
    dstw = dst_p.reshape(NW, NCH_DEG, CH)
    return _run(x, srcw, dsts, dstw, W1, b1.reshape(1, D), W2,
                b2.reshape(1, D), W3, b3.reshape(1, D), Wl, bl.reshape(1, 1))
